# Initial kernel scaffold; baseline (speedup 1.0000x reference)
import jax
import jax.numpy as jnp
from jax.experimental import pallas as pl


def kernel(x, ei, ea, params):
    raise NotImplementedError("write your pallas kernel here")


# speedup vs baseline: 16.4786x; 16.4786x over previous
import functools

import jax
import jax.numpy as jnp
from jax import lax
from jax.experimental import pallas as pl
from jax.experimental.pallas import tpu as pltpu
from jax.experimental.pallas import tpu_sc as plsc

N = 1024
E = 65536
HID = 64
EMB = 32

_SC_PARAMS = pltpu.CompilerParams(needs_layout_passes=False,
                                  use_tc_tiling_on_sc=False)


def _sc_mesh():
    return plsc.VectorSubcoreMesh(core_axis_name="c", subcore_axis_name="s")


def _make_pass_a(heads, C):
    n_slots = 32 // heads
    Ec = E // n_slots
    G = Ec // 16

    @functools.partial(
        pl.kernel,
        out_type=(jax.ShapeDtypeStruct((heads, E), jnp.float32),
                  jax.ShapeDtypeStruct((heads, n_slots, N), jnp.float32)),
        mesh=_sc_mesh(),
        scratch_types=[
            pltpu.VMEM((N, C), jnp.float32),
            pltpu.VMEM((N, C), jnp.float32),
            pltpu.VMEM((Ec,), jnp.int32),
            pltpu.VMEM((Ec,), jnp.int32),
            pltpu.VMEM((Ec,), jnp.float32),
            pltpu.VMEM((Ec,), jnp.float32),
            pltpu.VMEM((N,), jnp.float32),
            pltpu.VMEM((C, 16), jnp.float32),
            pltpu.VMEM((C, 16), jnp.float32),
        ],
        compiler_params=_SC_PARAMS,
    )
    def pass_a(xl_hbm, xr_hbm, src_hbm, dst_hbm, ea_hbm, we_hbm, att_hbm,
               ex_hbm, den_hbm,
               xl_v, xr_v, src_v, dst_v, ea_v, ex_v, den_v, we_v, att_v):
        cid = lax.axis_index("c")
        sid = lax.axis_index("s")
        w = sid * 2 + cid
        h = w // n_slots
        t = lax.rem(w, n_slots)
        base = t * Ec
        pltpu.sync_copy(xl_hbm.at[h], xl_v)
        pltpu.sync_copy(xr_hbm.at[h], xr_v)
        pltpu.sync_copy(src_hbm.at[pl.ds(base, Ec)], src_v)
        pltpu.sync_copy(dst_hbm.at[pl.ds(base, Ec)], dst_v)
        pltpu.sync_copy(ea_hbm.at[pl.ds(base, Ec)], ea_v)
        pltpu.sync_copy(we_hbm.at[h], we_v)
        pltpu.sync_copy(att_hbm.at[h], att_v)

        def zero(i, carry):
            den_v[pl.ds(i * 16, 16)] = jnp.zeros((16,), jnp.float32)
            return carry
        lax.fori_loop(0, N // 16, zero, None)

        def body(g, carry):
            off = g * 16
            s16 = src_v[pl.ds(off, 16)]
            d16 = dst_v[pl.ds(off, 16)]
            a16 = ea_v[pl.ds(off, 16)]
            acc = jnp.zeros((16,), jnp.float32)
            for c in range(C):
                cv = jnp.full((16,), c, jnp.int32)
                xlc = plsc.load_gather(xl_v, [s16, cv])
                xrc = plsc.load_gather(xr_v, [d16, cv])
                m = xlc + xrc + a16 * we_v[c]
                m = jnp.maximum(m, 0.2 * m)
                acc = acc + att_v[c] * m
            ex = jnp.exp(acc)
            ex_v[pl.ds(off, 16)] = ex
            plsc.addupdate_scatter(den_v, [d16], ex)
            return carry
        lax.fori_loop(0, G, body, None)

        pltpu.sync_copy(ex_v, ex_hbm.at[h, pl.ds(base, Ec)])
        pltpu.sync_copy(den_v, den_hbm.at[h, t])

    return pass_a


def _make_pass_b(heads, C):
    n_slots = 32 // heads
    Ec = E // n_slots
    G = Ec // 16

    @functools.partial(
        pl.kernel,
        out_type=jax.ShapeDtypeStruct((heads, n_slots, N, C), jnp.float32),
        mesh=_sc_mesh(),
        scratch_types=[
            pltpu.VMEM((N, C), jnp.float32),
            pltpu.VMEM((Ec,), jnp.int32),
            pltpu.VMEM((Ec,), jnp.int32),
            pltpu.VMEM((Ec,), jnp.float32),
            pltpu.VMEM((n_slots, N), jnp.float32),
            pltpu.VMEM((N,), jnp.float32),
            pltpu.VMEM((N, C), jnp.float32),
        ],
        compiler_params=_SC_PARAMS,
    )
    def pass_b(xl_hbm, src_hbm, dst_hbm, ex_hbm, den_hbm,
               out_hbm,
               xl_v, src_v, dst_v, ex_v, denp_v, den_v, out_v):
        cid = lax.axis_index("c")
        sid = lax.axis_index("s")
        w = sid * 2 + cid
        h = w // n_slots
        t = lax.rem(w, n_slots)
        base = t * Ec
        pltpu.sync_copy(xl_hbm.at[h], xl_v)
        pltpu.sync_copy(src_hbm.at[pl.ds(base, Ec)], src_v)
        pltpu.sync_copy(dst_hbm.at[pl.ds(base, Ec)], dst_v)
        pltpu.sync_copy(ex_hbm.at[h, pl.ds(base, Ec)], ex_v)
        pltpu.sync_copy(den_hbm.at[h], denp_v)

        def red(i, carry):
            off = i * 16
            acc = denp_v[0, pl.ds(off, 16)]
            for tt in range(1, n_slots):
                acc = acc + denp_v[tt, pl.ds(off, 16)]
            den_v[pl.ds(off, 16)] = acc
            return carry
        lax.fori_loop(0, N // 16, red, None)

        def zero(i, carry):
            for j in range(C // 16):
                out_v[i, pl.ds(j * 16, 16)] = jnp.zeros((16,), jnp.float32)
            return carry
        lax.fori_loop(0, N, zero, None)

        def body(g, carry):
            off = g * 16
            s16 = src_v[pl.ds(off, 16)]
            d16 = dst_v[pl.ds(off, 16)]
            ex16 = ex_v[pl.ds(off, 16)]
            den16 = plsc.load_gather(den_v, [d16])
            alpha = ex16 / (den16 + 1e-16)
            for c in range(C):
                cv = jnp.full((16,), c, jnp.int32)
                xlc = plsc.load_gather(xl_v, [s16, cv])
                plsc.addupdate_scatter(out_v, [d16, cv], xlc * alpha)
            return carry
        lax.fori_loop(0, G, body, None)

        pltpu.sync_copy(out_v, out_hbm.at[h, t])

    return pass_b


_pass_a_4 = _make_pass_a(4, 16)
_pass_b_4 = _make_pass_b(4, 16)
_pass_a_1 = _make_pass_a(1, 32)
_pass_b_1 = _make_pass_b(1, 32)


def _proj1_body(x_ref, wl_ref, bl_ref, wr_ref, br_ref, xl_out, xr_out):
    x = x_ref[...]
    xl = jnp.dot(x, wl_ref[...], preferred_element_type=jnp.float32) + bl_ref[...]
    xr = jnp.dot(x, wr_ref[...], preferred_element_type=jnp.float32) + br_ref[...]
    for h in range(4):
        xl_out[h] = xl[:, h * 16:(h + 1) * 16]
        xr_out[h] = xr[:, h * 16:(h + 1) * 16]


def _proj1(x, wl, bl, wr, br):
    return pl.pallas_call(
        _proj1_body,
        out_shape=(jax.ShapeDtypeStruct((4, N, 16), jnp.float32),
                   jax.ShapeDtypeStruct((4, N, 16), jnp.float32)),
    )(x, wl, bl, wr, br)


def _mid_body(part_ref, bias_ref, g_ref, b_ref, wl_ref, bl_ref, wr_ref, br_ref,
              xl_out, xr_out, *, n_slots, heads_next, c_next):
    acc = part_ref[0]
    for t in range(1, n_slots):
        acc = acc + part_ref[t]
    h = acc + bias_ref[...]
    m = jnp.mean(h, axis=-1, keepdims=True)
    v = jnp.mean((h - m) ** 2, axis=-1, keepdims=True)
    hn = (h - m) * lax.rsqrt(v + 1e-5) * g_ref[...] + b_ref[...]
    he = jnp.where(hn > 0, hn, jnp.exp(hn) - 1.0)
    xl = jnp.dot(he, wl_ref[...], preferred_element_type=jnp.float32) + bl_ref[...]
    xr = jnp.dot(he, wr_ref[...], preferred_element_type=jnp.float32) + br_ref[...]
    for hh in range(heads_next):
        xl_out[hh] = xl[:, hh * c_next:(hh + 1) * c_next]
        xr_out[hh] = xr[:, hh * c_next:(hh + 1) * c_next]


def _mid(part, bias, g, b, wl, bl, wr, br, heads_next, c_next):
    n_slots = part.shape[0]
    body = functools.partial(_mid_body, n_slots=n_slots,
                             heads_next=heads_next, c_next=c_next)
    return pl.pallas_call(
        body,
        out_shape=(jax.ShapeDtypeStruct((heads_next, N, c_next), jnp.float32),
                   jax.ShapeDtypeStruct((heads_next, N, c_next), jnp.float32)),
    )(part, bias, g, b, wl, bl, wr, br)


def _dec_pre_body(part_ref, bias_ref, g3_ref, b3_ref,
                  w1_ref, b1_ref, g_ref, w2_ref,
                  p_out, q_out, pg_out, vp_out, vq_out, sp_out, sq_out,
                  *, n_slots):
    acc = part_ref[0]
    for t in range(1, n_slots):
        acc = acc + part_ref[t]
    h = acc + bias_ref[...]
    m = jnp.mean(h, axis=-1, keepdims=True)
    v = jnp.mean((h - m) ** 2, axis=-1, keepdims=True)
    emb = (h - m) * lax.rsqrt(v + 1e-5) * g3_ref[...] + b3_ref[...]
    for p in range(4):
        w1 = w1_ref[p]
        a = jnp.dot(emb, w1[:EMB, :], preferred_element_type=jnp.float32) \
            + b1_ref[p]
        bq = jnp.dot(emb, w1[EMB:, :], preferred_element_type=jnp.float32)
        ma = jnp.mean(a, axis=-1, keepdims=True)
        mb = jnp.mean(bq, axis=-1, keepdims=True)
        pc = a - ma
        qc = bq - mb
        g = g_ref[p]
        w2 = w2_ref[p]
        wg = w2 * g
        p_out[p] = pc
        q_out[p] = qc
        pg_out[p] = pc * g[None, :]
        vp_out[p] = jnp.mean(pc * pc, axis=-1)
        vq_out[p] = jnp.mean(qc * qc, axis=-1)
        sp_out[p] = jnp.sum(pc * wg[None, :], axis=-1)
        sq_out[p] = jnp.sum(qc * wg[None, :], axis=-1)


def _dec_pre(part, bias, g3, b3, w1, b1, g, w2):
    n_slots = part.shape[0]
    body = functools.partial(_dec_pre_body, n_slots=n_slots)
    return pl.pallas_call(
        body,
        out_shape=(jax.ShapeDtypeStruct((4, N, EMB), jnp.float32),
                   jax.ShapeDtypeStruct((4, N, EMB), jnp.float32),
                   jax.ShapeDtypeStruct((4, N, EMB), jnp.float32),
                   jax.ShapeDtypeStruct((4, N), jnp.float32),
                   jax.ShapeDtypeStruct((4, N), jnp.float32),
                   jax.ShapeDtypeStruct((4, N), jnp.float32),
                   jax.ShapeDtypeStruct((4, N), jnp.float32),
                   ),
    )(part, bias, g3, b3, w1, b1, g, w2)


_IB = 16


def _dec_main_body(p_ref, qt_ref, pg_ref, qgt_ref, vp_ref, vq_ref,
                   sp_ref, sq_ref, bn_ref, w2_ref, b2_ref, out_ref):
    p_blk = p_ref[0]
    qt = qt_ref[0]
    pg = pg_ref[0]
    qgt = qgt_ref[0]
    vp = vp_ref[0]
    vq = vq_ref[0]
    sp = sp_ref[0]
    sq = sq_ref[0]
    bn = bn_ref[0]
    w2 = w2_ref[0]
    b2 = b2_ref[0]

    cross = jnp.dot(p_blk, qt, preferred_element_type=jnp.float32)
    var = vp + vq + (2.0 / EMB) * cross
    var = jnp.maximum(var, 0.0)
    r = lax.rsqrt(var + 1e-5)

    acc = jnp.zeros((_IB, N), jnp.float32)
    for c in range(EMB):
        t = pg[:, c:c + 1] + qgt[c:c + 1, :]
        z = t * r + bn[:, c:c + 1]
        acc = acc + jnp.abs(z) * w2[:, c:c + 1]
    cbn = jnp.sum(w2 * bn)
    lin = r * (sp + sq)
    out_ref[0] = 0.55 * lin + 0.45 * acc + (0.55 * cbn + b2[0, 0])


def _dec_main(p, qt, pg, qgt, vp, vq, sp, sq, bn, w2, b2):
    grid = (4, N // _IB)
    return pl.pallas_call(
        _dec_main_body,
        grid=grid,
        in_specs=[
            pl.BlockSpec((1, _IB, EMB), lambda p_, i: (p_, i, 0)),
            pl.BlockSpec((1, EMB, N), lambda p_, i: (p_, 0, 0)),
            pl.BlockSpec((1, _IB, EMB), lambda p_, i: (p_, i, 0)),
            pl.BlockSpec((1, EMB, N), lambda p_, i: (p_, 0, 0)),
            pl.BlockSpec((1, _IB, 1), lambda p_, i: (p_, i, 0)),
            pl.BlockSpec((1, 1, N), lambda p_, i: (p_, 0, 0)),
            pl.BlockSpec((1, _IB, 1), lambda p_, i: (p_, i, 0)),
            pl.BlockSpec((1, 1, N), lambda p_, i: (p_, 0, 0)),
            pl.BlockSpec((1, 1, EMB), lambda p_, i: (p_, 0, 0)),
            pl.BlockSpec((1, 1, EMB), lambda p_, i: (p_, 0, 0)),
            pl.BlockSpec((1, 1, 1), lambda p_, i: (p_, 0, 0)),
        ],
        out_specs=pl.BlockSpec((1, _IB, N), lambda p_, i: (p_, i, 0)),
        out_shape=jax.ShapeDtypeStruct((4, N, N), jnp.float32),
    )(p, qt, pg, qgt, vp, vq, sp, sq, bn, w2, b2)


def _splat(v):
    return jnp.broadcast_to(v[:, :, None], v.shape + (16,))


def _gat_layer(xl_hm, xr_hm, src, dst, ea, we_hm, att_hm, heads, C):
    pass_a = _pass_a_4 if heads == 4 else _pass_a_1
    pass_b = _pass_b_4 if heads == 4 else _pass_b_1
    ex, den = pass_a(xl_hm, xr_hm, src, dst, ea, _splat(we_hm), _splat(att_hm))
    part = pass_b(xl_hm, src, dst, ex, den)
    part = jnp.transpose(part, (1, 2, 0, 3)).reshape(part.shape[1], N, heads * C)
    return part


def kernel(x, ei, ea, params):
    src = ei[0].astype(jnp.int32)
    dst = ei[1].astype(jnp.int32)
    eav = ea[:, 0]

    c1, c2, c3 = params['c1'], params['c2'], params['c3']
    dec = params['dec']

    xl1, xr1 = _proj1(x, c1['Wl'], c1['bl'], c1['Wr'], c1['br'])
    part1 = _gat_layer(xl1, xr1, src, dst, eav,
                       c1['We'].reshape(4, 16), c1['att'], 4, 16)

    xl2, xr2 = _mid(part1, c1['bias'], params['n1'][0], params['n1'][1],
                    c2['Wl'], c2['bl'], c2['Wr'], c2['br'], 4, 16)
    part2 = _gat_layer(xl2, xr2, src, dst, eav,
                       c2['We'].reshape(4, 16), c2['att'], 4, 16)

    xl3, xr3 = _mid(part2, c2['bias'], params['n2'][0], params['n2'][1],
                    c3['Wl'], c3['bl'], c3['Wr'], c3['br'], 1, 32)
    part3 = _gat_layer(xl3, xr3, src, dst, eav,
                       c3['We'].reshape(1, 32), c3['att'], 1, 32)

    pmat, qmat, pgmat, vp, vq, sp, sq = _dec_pre(
        part3, c3['bias'], params['n3'][0], params['n3'][1],
        dec['W1'], dec['b1'], dec['g'], dec['W2'][:, :, 0])

    qt = jnp.transpose(qmat, (0, 2, 1))
    qgt = jnp.transpose(qmat * dec['g'][:, None, :], (0, 2, 1))
    out = _dec_main(pmat, qt, pgmat, qgt,
                    vp.reshape(4, N, 1), vq.reshape(4, 1, N),
                    sp.reshape(4, N, 1), sq.reshape(4, 1, N),
                    dec['bn'].reshape(4, 1, EMB),
                    dec['W2'][:, :, 0].reshape(4, 1, EMB),
                    dec['b2'].reshape(4, 1, 1))
    return jnp.transpose(out, (1, 2, 0))


# speedup vs baseline: 19.5187x; 1.1845x over previous
import functools

import jax
import jax.numpy as jnp
from jax import lax
from jax.experimental import pallas as pl
from jax.experimental.pallas import tpu as pltpu
from jax.experimental.pallas import tpu_sc as plsc

N = 1024
E = 65536
HID = 64
EMB = 32

_SC_PARAMS = pltpu.CompilerParams(needs_layout_passes=False,
                                  use_tc_tiling_on_sc=False)


def _sc_mesh():
    return plsc.VectorSubcoreMesh(core_axis_name="c", subcore_axis_name="s")


def _make_pass_a(heads, C):
    n_slots = 32 // heads
    Ec = E // n_slots
    G = Ec // 16

    @functools.partial(
        pl.kernel,
        out_type=(jax.ShapeDtypeStruct((heads, E), jnp.float32),
                  jax.ShapeDtypeStruct((heads, n_slots, N), jnp.float32)),
        mesh=_sc_mesh(),
        scratch_types=[
            pltpu.VMEM((N, C + 1), jnp.float32),
            pltpu.VMEM((N, C + 1), jnp.float32),
            pltpu.VMEM((Ec,), jnp.int32),
            pltpu.VMEM((Ec,), jnp.int32),
            pltpu.VMEM((Ec,), jnp.float32),
            pltpu.VMEM((Ec,), jnp.float32),
            pltpu.VMEM((N,), jnp.float32),
            pltpu.VMEM((C, 16), jnp.float32),
            pltpu.VMEM((C, 16), jnp.float32),
        ],
        compiler_params=_SC_PARAMS,
    )
    def pass_a(xl_hbm, xr_hbm, src_hbm, dst_hbm, ea_hbm, we_hbm, att_hbm,
               ex_hbm, den_hbm,
               xl_v, xr_v, src_v, dst_v, ea_v, ex_v, den_v, we_v, att_v):
        cid = lax.axis_index("c")
        sid = lax.axis_index("s")
        w = sid * 2 + cid
        h = w // n_slots
        t = lax.rem(w, n_slots)
        base = t * Ec
        pltpu.sync_copy(xl_hbm.at[h], xl_v)
        pltpu.sync_copy(xr_hbm.at[h], xr_v)
        pltpu.sync_copy(src_hbm.at[pl.ds(base, Ec)], src_v)
        pltpu.sync_copy(dst_hbm.at[pl.ds(base, Ec)], dst_v)
        pltpu.sync_copy(ea_hbm.at[pl.ds(base, Ec)], ea_v)
        pltpu.sync_copy(we_hbm.at[h], we_v)
        pltpu.sync_copy(att_hbm.at[h], att_v)

        def zero(i, carry):
            den_v[pl.ds(i * 16, 16)] = jnp.zeros((16,), jnp.float32)
            return carry
        lax.fori_loop(0, N // 16, zero, None)

        def body(g, carry):
            off = g * 16
            s16 = src_v[pl.ds(off, 16)]
            d16 = dst_v[pl.ds(off, 16)]
            a16 = ea_v[pl.ds(off, 16)]
            acc = jnp.zeros((16,), jnp.float32)
            for c in range(C):
                cv = jnp.full((16,), c, jnp.int32)
                xlc = plsc.load_gather(xl_v, [s16, cv])
                xrc = plsc.load_gather(xr_v, [d16, cv])
                m = xlc + xrc + a16 * we_v[c]
                m = jnp.maximum(m, 0.2 * m)
                acc = acc + att_v[c] * m
            ex = jnp.exp(acc)
            ex_v[pl.ds(off, 16)] = ex
            plsc.addupdate_scatter(den_v, [d16], ex)
            return carry
        lax.fori_loop(0, G, body, None)

        pltpu.sync_copy(ex_v, ex_hbm.at[h, pl.ds(base, Ec)])
        pltpu.sync_copy(den_v, den_hbm.at[h, t])

    return pass_a


def _make_pass_b(heads, C):
    n_slots = 32 // heads
    Ec = E // n_slots
    G = Ec // 16

    @functools.partial(
        pl.kernel,
        out_type=jax.ShapeDtypeStruct((heads, n_slots, N, C + 1), jnp.float32),
        mesh=_sc_mesh(),
        scratch_types=[
            pltpu.VMEM((N, C + 1), jnp.float32),
            pltpu.VMEM((Ec,), jnp.int32),
            pltpu.VMEM((Ec,), jnp.int32),
            pltpu.VMEM((Ec,), jnp.float32),
            pltpu.VMEM((n_slots, N), jnp.float32),
            pltpu.VMEM((N,), jnp.float32),
            pltpu.VMEM((N, C + 1), jnp.float32),
        ],
        compiler_params=_SC_PARAMS,
    )
    def pass_b(xl_hbm, src_hbm, dst_hbm, ex_hbm, den_hbm,
               out_hbm,
               xl_v, src_v, dst_v, ex_v, denp_v, den_v, out_v):
        cid = lax.axis_index("c")
        sid = lax.axis_index("s")
        w = sid * 2 + cid
        h = w // n_slots
        t = lax.rem(w, n_slots)
        base = t * Ec
        pltpu.sync_copy(xl_hbm.at[h], xl_v)
        pltpu.sync_copy(src_hbm.at[pl.ds(base, Ec)], src_v)
        pltpu.sync_copy(dst_hbm.at[pl.ds(base, Ec)], dst_v)
        pltpu.sync_copy(ex_hbm.at[h, pl.ds(base, Ec)], ex_v)
        pltpu.sync_copy(den_hbm.at[h], denp_v)

        def red(i, carry):
            off = i * 16
            acc = denp_v[0, pl.ds(off, 16)]
            for tt in range(1, n_slots):
                acc = acc + denp_v[tt, pl.ds(off, 16)]
            den_v[pl.ds(off, 16)] = acc
            return carry
        lax.fori_loop(0, N // 16, red, None)

        def zero(i, carry):
            for j in range(C // 16):
                out_v[i, pl.ds(j * 16, 16)] = jnp.zeros((16,), jnp.float32)
            return carry
        lax.fori_loop(0, N, zero, None)

        def body(g, carry):
            off = g * 16
            s16 = src_v[pl.ds(off, 16)]
            d16 = dst_v[pl.ds(off, 16)]
            ex16 = ex_v[pl.ds(off, 16)]
            den16 = plsc.load_gather(den_v, [d16])
            alpha = ex16 / (den16 + 1e-16)
            for c in range(C):
                cv = jnp.full((16,), c, jnp.int32)
                xlc = plsc.load_gather(xl_v, [s16, cv])
                plsc.addupdate_scatter(out_v, [d16, cv], xlc * alpha)
            return carry
        lax.fori_loop(0, G, body, None)

        pltpu.sync_copy(out_v, out_hbm.at[h, t])

    return pass_b


_pass_a_4 = _make_pass_a(4, 16)
_pass_b_4 = _make_pass_b(4, 16)
_pass_a_1 = _make_pass_a(1, 32)
_pass_b_1 = _make_pass_b(1, 32)


def _proj1_body(x_ref, wl_ref, bl_ref, wr_ref, br_ref, xl_out, xr_out):
    x = x_ref[...]
    xl = jnp.dot(x, wl_ref[...], preferred_element_type=jnp.float32) + bl_ref[...]
    xr = jnp.dot(x, wr_ref[...], preferred_element_type=jnp.float32) + br_ref[...]
    for h in range(4):
        xl_out[h] = xl[:, h * 16:(h + 1) * 16]
        xr_out[h] = xr[:, h * 16:(h + 1) * 16]


def _proj1(x, wl, bl, wr, br):
    return pl.pallas_call(
        _proj1_body,
        out_shape=(jax.ShapeDtypeStruct((4, N, 16), jnp.float32),
                   jax.ShapeDtypeStruct((4, N, 16), jnp.float32)),
    )(x, wl, bl, wr, br)


def _mid_body(part_ref, bias_ref, g_ref, b_ref, wl_ref, bl_ref, wr_ref, br_ref,
              xl_out, xr_out, *, n_slots, heads_next, c_next):
    acc = part_ref[0]
    for t in range(1, n_slots):
        acc = acc + part_ref[t]
    h = acc + bias_ref[...]
    m = jnp.mean(h, axis=-1, keepdims=True)
    v = jnp.mean((h - m) ** 2, axis=-1, keepdims=True)
    hn = (h - m) * lax.rsqrt(v + 1e-5) * g_ref[...] + b_ref[...]
    he = jnp.where(hn > 0, hn, jnp.exp(hn) - 1.0)
    xl = jnp.dot(he, wl_ref[...], preferred_element_type=jnp.float32) + bl_ref[...]
    xr = jnp.dot(he, wr_ref[...], preferred_element_type=jnp.float32) + br_ref[...]
    for hh in range(heads_next):
        xl_out[hh] = xl[:, hh * c_next:(hh + 1) * c_next]
        xr_out[hh] = xr[:, hh * c_next:(hh + 1) * c_next]


def _mid(part, bias, g, b, wl, bl, wr, br, heads_next, c_next):
    n_slots = part.shape[0]
    body = functools.partial(_mid_body, n_slots=n_slots,
                             heads_next=heads_next, c_next=c_next)
    return pl.pallas_call(
        body,
        out_shape=(jax.ShapeDtypeStruct((heads_next, N, c_next), jnp.float32),
                   jax.ShapeDtypeStruct((heads_next, N, c_next), jnp.float32)),
    )(part, bias, g, b, wl, bl, wr, br)


def _dec_pre_body(part_ref, bias_ref, g3_ref, b3_ref,
                  w1_ref, b1_ref, g_ref, w2_ref,
                  p_out, q_out, pg_out, vp_out, vq_out, sp_out, sq_out,
                  *, n_slots):
    acc = part_ref[0]
    for t in range(1, n_slots):
        acc = acc + part_ref[t]
    h = acc + bias_ref[...]
    m = jnp.mean(h, axis=-1, keepdims=True)
    v = jnp.mean((h - m) ** 2, axis=-1, keepdims=True)
    emb = (h - m) * lax.rsqrt(v + 1e-5) * g3_ref[...] + b3_ref[...]
    for p in range(4):
        w1 = w1_ref[p]
        a = jnp.dot(emb, w1[:EMB, :], preferred_element_type=jnp.float32) \
            + b1_ref[p]
        bq = jnp.dot(emb, w1[EMB:, :], preferred_element_type=jnp.float32)
        ma = jnp.mean(a, axis=-1, keepdims=True)
        mb = jnp.mean(bq, axis=-1, keepdims=True)
        pc = a - ma
        qc = bq - mb
        g = g_ref[p]
        w2 = w2_ref[p]
        wg = w2 * g
        p_out[p] = pc
        q_out[p] = qc
        pg_out[p] = pc * g[None, :]
        vp_out[p] = jnp.mean(pc * pc, axis=-1)
        vq_out[p] = jnp.mean(qc * qc, axis=-1)
        sp_out[p] = jnp.sum(pc * wg[None, :], axis=-1)
        sq_out[p] = jnp.sum(qc * wg[None, :], axis=-1)


def _dec_pre(part, bias, g3, b3, w1, b1, g, w2):
    n_slots = part.shape[0]
    body = functools.partial(_dec_pre_body, n_slots=n_slots)
    return pl.pallas_call(
        body,
        out_shape=(jax.ShapeDtypeStruct((4, N, EMB), jnp.float32),
                   jax.ShapeDtypeStruct((4, N, EMB), jnp.float32),
                   jax.ShapeDtypeStruct((4, N, EMB), jnp.float32),
                   jax.ShapeDtypeStruct((4, N), jnp.float32),
                   jax.ShapeDtypeStruct((4, N), jnp.float32),
                   jax.ShapeDtypeStruct((4, N), jnp.float32),
                   jax.ShapeDtypeStruct((4, N), jnp.float32),
                   ),
    )(part, bias, g3, b3, w1, b1, g, w2)


_IB = 16


def _dec_main_body(p_ref, qt_ref, pg_ref, qgt_ref, vp_ref, vq_ref,
                   sp_ref, sq_ref, bn_ref, w2_ref, b2_ref, out_ref):
    p_blk = p_ref[0]
    qt = qt_ref[0]
    pg = pg_ref[0]
    qgt = qgt_ref[0]
    vp = vp_ref[0]
    vq = vq_ref[0]
    sp = sp_ref[0]
    sq = sq_ref[0]
    bn = bn_ref[0]
    w2 = w2_ref[0]
    b2 = b2_ref[0]

    cross = jnp.dot(p_blk, qt, preferred_element_type=jnp.float32)
    var = vp + vq + (2.0 / EMB) * cross
    var = jnp.maximum(var, 0.0)
    r = lax.rsqrt(var + 1e-5)

    acc = jnp.zeros((_IB, N), jnp.float32)
    for c in range(EMB):
        t = pg[:, c:c + 1] + qgt[c:c + 1, :]
        z = t * r + bn[:, c:c + 1]
        acc = acc + jnp.abs(z) * w2[:, c:c + 1]
    cbn = jnp.sum(w2 * bn)
    lin = r * (sp + sq)
    out_ref[0] = 0.55 * lin + 0.45 * acc + (0.55 * cbn + b2[0, 0])


def _dec_main(p, qt, pg, qgt, vp, vq, sp, sq, bn, w2, b2):
    grid = (4, N // _IB)
    return pl.pallas_call(
        _dec_main_body,
        grid=grid,
        in_specs=[
            pl.BlockSpec((1, _IB, EMB), lambda p_, i: (p_, i, 0)),
            pl.BlockSpec((1, EMB, N), lambda p_, i: (p_, 0, 0)),
            pl.BlockSpec((1, _IB, EMB), lambda p_, i: (p_, i, 0)),
            pl.BlockSpec((1, EMB, N), lambda p_, i: (p_, 0, 0)),
            pl.BlockSpec((1, _IB, 1), lambda p_, i: (p_, i, 0)),
            pl.BlockSpec((1, 1, N), lambda p_, i: (p_, 0, 0)),
            pl.BlockSpec((1, _IB, 1), lambda p_, i: (p_, i, 0)),
            pl.BlockSpec((1, 1, N), lambda p_, i: (p_, 0, 0)),
            pl.BlockSpec((1, 1, EMB), lambda p_, i: (p_, 0, 0)),
            pl.BlockSpec((1, 1, EMB), lambda p_, i: (p_, 0, 0)),
            pl.BlockSpec((1, 1, 1), lambda p_, i: (p_, 0, 0)),
        ],
        out_specs=pl.BlockSpec((1, _IB, N), lambda p_, i: (p_, i, 0)),
        out_shape=jax.ShapeDtypeStruct((4, N, N), jnp.float32),
    )(p, qt, pg, qgt, vp, vq, sp, sq, bn, w2, b2)


def _splat(v):
    return jnp.broadcast_to(v[:, :, None], v.shape + (16,))


def _gat_layer(xl_hm, xr_hm, src, dst, ea, we_hm, att_hm, heads, C):
    pass_a = _pass_a_4 if heads == 4 else _pass_a_1
    pass_b = _pass_b_4 if heads == 4 else _pass_b_1
    pad = jnp.zeros((heads, N, 1), jnp.float32)
    xlp = jnp.concatenate([xl_hm, pad], axis=-1)
    xrp = jnp.concatenate([xr_hm, pad], axis=-1)
    ex, den = pass_a(xlp, xrp, src, dst, ea, _splat(we_hm), _splat(att_hm))
    part = pass_b(xlp, src, dst, ex, den)[..., :C]
    part = jnp.transpose(part, (1, 2, 0, 3)).reshape(part.shape[1], N, heads * C)
    return part


def kernel(x, ei, ea, params):
    src = ei[0].astype(jnp.int32)
    dst = ei[1].astype(jnp.int32)
    eav = ea[:, 0]

    c1, c2, c3 = params['c1'], params['c2'], params['c3']
    dec = params['dec']

    xl1, xr1 = _proj1(x, c1['Wl'], c1['bl'], c1['Wr'], c1['br'])
    part1 = _gat_layer(xl1, xr1, src, dst, eav,
                       c1['We'].reshape(4, 16), c1['att'], 4, 16)

    xl2, xr2 = _mid(part1, c1['bias'], params['n1'][0], params['n1'][1],
                    c2['Wl'], c2['bl'], c2['Wr'], c2['br'], 4, 16)
    part2 = _gat_layer(xl2, xr2, src, dst, eav,
                       c2['We'].reshape(4, 16), c2['att'], 4, 16)

    xl3, xr3 = _mid(part2, c2['bias'], params['n2'][0], params['n2'][1],
                    c3['Wl'], c3['bl'], c3['Wr'], c3['br'], 1, 32)
    part3 = _gat_layer(xl3, xr3, src, dst, eav,
                       c3['We'].reshape(1, 32), c3['att'], 1, 32)

    pmat, qmat, pgmat, vp, vq, sp, sq = _dec_pre(
        part3, c3['bias'], params['n3'][0], params['n3'][1],
        dec['W1'], dec['b1'], dec['g'], dec['W2'][:, :, 0])

    qt = jnp.transpose(qmat, (0, 2, 1))
    qgt = jnp.transpose(qmat * dec['g'][:, None, :], (0, 2, 1))
    out = _dec_main(pmat, qt, pgmat, qgt,
                    vp.reshape(4, N, 1), vq.reshape(4, 1, N),
                    sp.reshape(4, N, 1), sq.reshape(4, 1, N),
                    dec['bn'].reshape(4, 1, EMB),
                    dec['W2'][:, :, 0].reshape(4, 1, EMB),
                    dec['b2'].reshape(4, 1, 1))
    return jnp.transpose(out, (1, 2, 0))
